# Initial kernel scaffold; baseline (speedup 1.0000x reference)
import jax
import jax.numpy as jnp
from jax.experimental import pallas as pl


def kernel(x):
    raise NotImplementedError("write your pallas kernel here")


# speedup vs baseline: 1.0087x; 1.0087x over previous
import jax
import jax.numpy as jnp
from jax.experimental import pallas as pl
from jax.experimental.pallas import tpu as pltpu

_PROB = jnp.array([0.25, 0.25, 0.5], dtype=jnp.float32)


def _switch_affine_kernel(idx_ref, x_ref, o_ref):
    idx = idx_ref[0]
    a = jnp.where(idx == 0, 2.0, jnp.where(idx == 1, 1.0, -1.0)).astype(jnp.float32)
    b = jnp.where(idx == 1, 1.0, 0.0).astype(jnp.float32)
    o_ref[...] = x_ref[...] * a + b


def kernel(x):
    idx = jax.random.categorical(jax.random.key(42), jnp.log(_PROB)).astype(jnp.int32)

    shape = x.shape
    x2 = x.reshape(-1, shape[-1])
    rows, cols = x2.shape
    block_rows = 2048
    grid = rows // block_rows

    out = pl.pallas_call(
        _switch_affine_kernel,
        grid_spec=pltpu.PrefetchScalarGridSpec(
            num_scalar_prefetch=1,
            grid=(grid,),
            in_specs=[pl.BlockSpec((block_rows, cols), lambda i, s: (i, 0))],
            out_specs=pl.BlockSpec((block_rows, cols), lambda i, s: (i, 0)),
        ),
        out_shape=jax.ShapeDtypeStruct(x2.shape, x2.dtype),
    )(idx.reshape(1), x2)
    return out.reshape(shape)
